# baseline (speedup 1.0000x reference)
import functools

import jax
import jax.numpy as jnp
import numpy as np
from jax import lax
from jax.experimental import pallas as pl
from jax.experimental.pallas import tpu as pltpu
from jax.experimental.pallas import tpu_sc as plsc

D_MODEL = 256
N_LEVELS = 4
N_HEADS = 8
N_POINTS = 4
SHAPES_PY = [(64, 64), (32, 32), (16, 16), (8, 8)]
LEN_IN = sum(h * w for h, w in SHAPES_PY)
LSI = [0]
for _h, _w in SHAPES_PY[:-1]:
    LSI.append(LSI[-1] + _h * _w)

BT = 680
BT2 = 512
BQ = 512
LQ_PAD = 5632
NCHUNK = 32


def _consts():
    M, L, P = N_HEADS, N_LEVELS, N_POINTS
    sel_rp = np.zeros((256, 8), np.float32)
    wh256 = np.zeros((256, 1), np.float32)
    for u in range(256):
        m, rem = divmod(u, 32)
        lp, comp = divmod(rem, 2)
        l = lp // P
        sel_rp[u, l * 2 + comp] = 1.0
        wh256[u, 0] = float(SHAPES_PY[l][1] if comp == 0 else SHAPES_PY[l][0])
    sel_x = np.zeros((512, 256), np.float32)
    sel_y = np.zeros((512, 256), np.float32)
    sel_a = np.zeros((512, 128), np.float32)
    dxv = np.zeros((512, 1), np.float32)
    dyv = np.zeros((512, 1), np.float32)
    wv = np.zeros((512, 1), np.float32)
    hv = np.zeros((512, 1), np.float32)
    lsiv = np.zeros((512, 1), np.float32)
    for r in range(512):
        m, rem = divmod(r, 64)
        lp, c4 = divmod(rem, 4)
        l = lp // P
        sel_x[r, m * 32 + lp * 2] = 1.0
        sel_y[r, m * 32 + lp * 2 + 1] = 1.0
        sel_a[r, m * 16 + lp] = 1.0
        dxv[r, 0] = float(c4 % 2)
        dyv[r, 0] = float(c4 // 2)
        wv[r, 0] = float(SHAPES_PY[l][1])
        hv[r, 0] = float(SHAPES_PY[l][0])
        lsiv[r, 0] = float(LSI[l])
    bd = np.kron(np.eye(N_HEADS, dtype=np.float32), np.ones((16, 16), np.float32))
    return tuple(jnp.asarray(a) for a in
                 (sel_rp, wh256, sel_x, sel_y, sel_a, dxv, dyv, wv, hv, lsiv, bd))


def _bf16_bits(x):
    b = jax.lax.bitcast_convert_type(x, jnp.int32)
    return (b + 0x7FFF + ((b >> 16) & 1)) >> 16


def _bf16_sbits(x):
    b = jax.lax.bitcast_convert_type(x, jnp.int32)
    return ((b + 0x7FFF + ((b >> 16) & 1)) >> 16) & 0xFFFF


def _prelude_body(qt_ref, xt_ref, rpt_ref, wo_ref, bo_ref, wa_ref, ba_ref,
                  wve_ref, bve_ref, wvo_ref, bvo_ref,
                  selrp_ref, wh_ref, selx_ref, sely_ref, sela_ref,
                  dx_ref, dy_ref, wv_ref, hv_ref, lsi_ref, bd_ref,
                  iw_ref, val_ref):
    qt = qt_ref[0]
    xt = xt_ref[0]
    rpt = rpt_ref[0]
    dot = lambda a, b: jnp.dot(a, b, preferred_element_type=jnp.float32,
                               precision=jax.lax.Precision.HIGHEST)

    vlo = dot(wve_ref[...], xt) + bve_ref[...]
    vhi = dot(wvo_ref[...], xt) + bvo_ref[...]
    val_ref[0] = _bf16_sbits(vlo) | (_bf16_sbits(vhi) << 16)

    offt = dot(wo_ref[...], qt) + bo_ref[...]
    z = dot(wa_ref[...], qt) + ba_ref[...]
    e = jnp.exp(z)
    attn = e / dot(bd_ref[...], e)

    g = dot(selrp_ref[...], rpt) * wh_ref[...] + offt - 0.5
    gx = dot(selx_ref[...], g)
    gy = dot(sely_ref[...], g)
    av = dot(sela_ref[...], attn)

    dx = dx_ref[...]
    dy = dy_ref[...]
    wl = wv_ref[...]
    hl = hv_ref[...]
    x0 = jnp.floor(gx)
    y0 = jnp.floor(gy)
    fx = gx - x0
    fy = gy - y0
    xf = x0 + dx
    yf = y0 + dy
    wx = dx * fx + (1.0 - dx) * (1.0 - fx)
    wy = dy * fy + (1.0 - dy) * (1.0 - fy)
    valid = ((xf >= 0.0) & (xf <= wl - 1.0) & (yf >= 0.0)
             & (yf <= hl - 1.0)).astype(jnp.float32)
    xi = jnp.clip(xf, 0.0, wl - 1.0)
    yi = jnp.clip(yf, 0.0, hl - 1.0)
    rows = (lsi_ref[...] + yi * wl + xi).astype(jnp.int32)
    w = av * wx * wy * valid
    iw_ref[0] = (rows << 16) | _bf16_bits(w)


def _tc_prelude(qt, xt, rpt, W_off, b_off, W_attn, b_attn, W_val, b_val):
    N = qt.shape[0]
    consts = _consts()
    grid = (N, LQ_PAD // BT2)
    full = lambda shape: pl.BlockSpec(shape, lambda n, i: (0,) * len(shape))
    blk = lambda r: pl.BlockSpec((1, r, BT2), lambda n, i: (n, 0, i))
    col = lambda v: v.reshape(-1, 1)
    return pl.pallas_call(
        _prelude_body,
        grid=grid,
        in_specs=[
            blk(256), blk(256), blk(8),
            full((256, 256)), full((256, 1)),
            full((128, 256)), full((128, 1)),
            full((128, 256)), full((128, 1)),
            full((128, 256)), full((128, 1)),
            full((256, 8)), full((256, 1)),
            full((512, 256)), full((512, 256)), full((512, 128)),
            full((512, 1)), full((512, 1)), full((512, 1)), full((512, 1)),
            full((512, 1)), full((128, 128)),
        ],
        out_specs=[blk(512), blk(128)],
        out_shape=[
            jax.ShapeDtypeStruct((N, 512, LQ_PAD), jnp.int32),
            jax.ShapeDtypeStruct((N, 128, LQ_PAD), jnp.int32),
        ],
    )(qt, xt, rpt,
      W_off.T, col(b_off), W_attn.T, col(b_attn),
      W_val[:, 0::2].T, col(b_val[0::2]), W_val[:, 1::2].T, col(b_val[1::2]),
      *consts)


def _proj_body(x_ref, w_ref, b_ref, o_ref):
    o_ref[0] = jnp.dot(x_ref[0], w_ref[...],
                       preferred_element_type=jnp.float32) + b_ref[...]


def _tc_out_proj(x, W_out, b_out):
    N, Lq, C = x.shape
    grid = (N, Lq // BT)
    return pl.pallas_call(
        _proj_body,
        grid=grid,
        in_specs=[
            pl.BlockSpec((1, BT, C), lambda n, i: (n, i, 0)),
            pl.BlockSpec((C, C), lambda n, i: (0, 0)),
            pl.BlockSpec((1, C), lambda n, i: (0, 0)),
        ],
        out_specs=pl.BlockSpec((1, BT, C), lambda n, i: (n, i, 0)),
        out_shape=jax.ShapeDtypeStruct((N, Lq, C), jnp.float32),
    )(x, W_out, b_out.reshape(1, -1))


def _sc_sample(val_t, iw_t):
    N = val_t.shape[0]
    nblk = LQ_PAD // BQ
    mesh = plsc.VectorSubcoreMesh(core_axis_name="c", subcore_axis_name="s")

    @functools.partial(
        pl.kernel,
        out_type=jax.ShapeDtypeStruct((NCHUNK, 16, LQ_PAD), jnp.float32),
        mesh=mesh,
        compiler_params=pltpu.CompilerParams(needs_layout_passes=False),
        scratch_types=[
            pltpu.VMEM((8 * LQ_PAD,), jnp.int32),
            pltpu.VMEM((64, BQ), jnp.int32),
            pltpu.VMEM((16, BQ), jnp.float32),
        ],
    )
    def sc_kernel(val_hbm, iw_hbm, out_hbm, tbl, iwb, outb):
        wid = lax.axis_index("s") * 2 + lax.axis_index("c")
        n = wid // 16
        cg = wid % 16
        m = cg // 2

        for p in range(8):
            pltpu.sync_copy(val_hbm.at[n, cg * 8 + p],
                            tbl.at[pl.ds(p * LQ_PAD, LQ_PAD)])

        def qb_body(qb, _):
            base = qb * BQ
            pltpu.sync_copy(iw_hbm.at[n, pl.ds(m * 64, 64), pl.ds(base, BQ)],
                            iwb)

            def g_body(g, _):
                def j_body(j, accs):
                    iwv = iwb[j, pl.ds(g * 16, 16)]
                    idxv = iwv >> 16
                    wv = plsc.bitcast(iwv << 16, jnp.float32)
                    new = []
                    for p in range(8):
                        vi = plsc.load_gather(tbl, [idxv + p * LQ_PAD])
                        lo = plsc.bitcast(vi << 16, jnp.float32)
                        hi = plsc.bitcast(vi & jnp.int32(-65536), jnp.float32)
                        new.append(accs[2 * p] + wv * lo)
                        new.append(accs[2 * p + 1] + wv * hi)
                    return tuple(new)

                accs = lax.fori_loop(
                    0, 64, j_body,
                    tuple(jnp.zeros((16,), jnp.float32) for _ in range(16)))
                for cc in range(16):
                    outb[cc, pl.ds(g * 16, 16)] = accs[cc]
                return 0

            lax.fori_loop(0, BQ // 16, g_body, 0)
            pltpu.sync_copy(outb, out_hbm.at[wid, :, pl.ds(base, BQ)])
            return 0

        lax.fori_loop(0, nblk, qb_body, 0)

    return sc_kernel(val_t, iw_t)


def kernel(query, reference_points, input_flatten, input_spatial_shapes,
           input_level_start_index, W_val, b_val, W_off, b_off,
           W_attn, b_attn, W_out, b_out):
    N, Lq, C = query.shape
    pad = ((0, 0), (0, 0), (0, LQ_PAD - Lq))
    qt = jnp.pad(query.transpose(0, 2, 1), pad)
    xt = jnp.pad(input_flatten.transpose(0, 2, 1), pad)
    rpt = jnp.pad(reference_points.reshape(N, Lq, 8).transpose(0, 2, 1), pad)
    iw_t, val_t = _tc_prelude(qt, xt, rpt, W_off, b_off, W_attn, b_attn,
                              W_val, b_val)
    out_t = _sc_sample(val_t, iw_t)
    sampled = out_t.reshape(N, 16, 16, LQ_PAD).transpose(0, 3, 1, 2)
    sampled = sampled.reshape(N, LQ_PAD, C)[:, :Lq]
    return _tc_out_proj(sampled, W_out, b_out)


# speedup vs baseline: 1.1308x; 1.1308x over previous
import functools

import jax
import jax.numpy as jnp
import numpy as np
from jax import lax
from jax.experimental import pallas as pl
from jax.experimental.pallas import tpu as pltpu
from jax.experimental.pallas import tpu_sc as plsc

D_MODEL = 256
N_LEVELS = 4
N_HEADS = 8
N_POINTS = 4
SHAPES_PY = [(64, 64), (32, 32), (16, 16), (8, 8)]
LEN_IN = sum(h * w for h, w in SHAPES_PY)
LSI = [0]
for _h, _w in SHAPES_PY[:-1]:
    LSI.append(LSI[-1] + _h * _w)

BT = 680
BT2 = 512
BQ = 512
LQ_PAD = 5632
NCHUNK = 32


def _consts():
    M, P = N_HEADS, N_POINTS
    sel_rp = np.zeros((256, 8), np.float32)
    wh256 = np.zeros((256, 1), np.float32)
    for u in range(256):
        comp, rem = divmod(u, 128)
        m, lp = divmod(rem, 16)
        l = lp // P
        sel_rp[u, l * 2 + comp] = 1.0
        wh256[u, 0] = float(SHAPES_PY[l][1] if comp == 0 else SHAPES_PY[l][0])
    wv = np.zeros((128, 1), np.float32)
    hv = np.zeros((128, 1), np.float32)
    lsiv = np.zeros((128, 1), np.float32)
    for r in range(128):
        l = (r % 16) // P
        wv[r, 0] = float(SHAPES_PY[l][1])
        hv[r, 0] = float(SHAPES_PY[l][0])
        lsiv[r, 0] = float(LSI[l])
    bd = np.kron(np.eye(N_HEADS, dtype=np.float32), np.ones((16, 16), np.float32))
    return tuple(jnp.asarray(a) for a in (sel_rp, wh256, wv, hv, lsiv, bd))


def _off_perm():
    perm = []
    for comp in range(2):
        for m in range(N_HEADS):
            for lp in range(16):
                perm.append(m * 32 + lp * 2 + comp)
    return np.asarray(perm, np.int64)


def _bf16_bits(x):
    b = jax.lax.bitcast_convert_type(x, jnp.int32)
    return (b + 0x7FFF + ((b >> 16) & 1)) >> 16


def _bf16_sbits(x):
    b = jax.lax.bitcast_convert_type(x, jnp.int32)
    return ((b + 0x7FFF + ((b >> 16) & 1)) >> 16) & 0xFFFF


def _prelude_body(qt_ref, xt_ref, rpt_ref, wo_ref, bo_ref, wa_ref, ba_ref,
                  wve_ref, bve_ref, wvo_ref, bvo_ref,
                  selrp_ref, wh_ref, wv_ref, hv_ref, lsi_ref, bd_ref,
                  iw_ref, val_ref):
    qt = qt_ref[0]
    xt = xt_ref[0]
    rpt = rpt_ref[0]
    dot = lambda a, b: jnp.dot(a, b, preferred_element_type=jnp.float32)

    vlo = dot(wve_ref[...], xt) + bve_ref[...]
    vhi = dot(wvo_ref[...], xt) + bvo_ref[...]
    val_ref[0] = _bf16_sbits(vlo) | (_bf16_sbits(vhi) << 16)

    offt = dot(wo_ref[...], qt) + bo_ref[...]
    z = dot(wa_ref[...], qt) + ba_ref[...]
    e = jnp.exp(z)
    av = e / dot(bd_ref[...], e)

    rpx = jnp.dot(selrp_ref[...], rpt, preferred_element_type=jnp.float32,
                  precision=jax.lax.Precision.HIGHEST)
    g = rpx * wh_ref[...] + offt - 0.5
    gx = g[:128]
    gy = g[128:]

    wl = wv_ref[...]
    hl = hv_ref[...]
    x0 = jnp.floor(gx)
    y0 = jnp.floor(gy)
    fx = gx - x0
    fy = gy - y0
    wx1 = fx
    wx0 = 1.0 - fx
    wy1 = fy
    wy0 = 1.0 - fy
    for c4, (dy, dx) in enumerate(((0., 0.), (0., 1.), (1., 0.), (1., 1.))):
        xf = x0 + dx
        yf = y0 + dy
        wx = wx1 if dx else wx0
        wy = wy1 if dy else wy0
        valid = ((xf >= 0.0) & (xf <= wl - 1.0) & (yf >= 0.0)
                 & (yf <= hl - 1.0)).astype(jnp.float32)
        xi = jnp.clip(xf, 0.0, wl - 1.0)
        yi = jnp.clip(yf, 0.0, hl - 1.0)
        rows = (lsi_ref[...] + yi * wl + xi).astype(jnp.int32)
        w = av * wx * wy * valid
        iw_ref[0, c4 * 128:(c4 + 1) * 128] = (rows << 16) | _bf16_bits(w)


def _tc_prelude(qt, xt, rpt, W_off, b_off, W_attn, b_attn, W_val, b_val):
    N = qt.shape[0]
    consts = _consts()
    grid = (N, LQ_PAD // BT2)
    full = lambda shape: pl.BlockSpec(shape, lambda n, i: (0,) * len(shape))
    blk = lambda r: pl.BlockSpec((1, r, BT2), lambda n, i: (n, 0, i))
    col = lambda v: v.reshape(-1, 1)
    return pl.pallas_call(
        _prelude_body,
        grid=grid,
        in_specs=[
            blk(256), blk(256), blk(8),
            full((256, 256)), full((256, 1)),
            full((128, 256)), full((128, 1)),
            full((128, 256)), full((128, 1)),
            full((128, 256)), full((128, 1)),
            full((256, 8)), full((256, 1)),
            full((128, 1)), full((128, 1)), full((128, 1)),
            full((128, 128)),
        ],
        out_specs=[blk(512), blk(128)],
        out_shape=[
            jax.ShapeDtypeStruct((N, 512, LQ_PAD), jnp.int32),
            jax.ShapeDtypeStruct((N, 128, LQ_PAD), jnp.int32),
        ],
    )(qt, xt, rpt,
      W_off[:, _off_perm()].T, col(b_off[_off_perm()]),
      W_attn.T, col(b_attn),
      W_val[:, 0::2].T, col(b_val[0::2]), W_val[:, 1::2].T, col(b_val[1::2]),
      *consts)


def _proj_body(x_ref, w_ref, b_ref, o_ref):
    o_ref[0] = jnp.dot(x_ref[0], w_ref[...],
                       preferred_element_type=jnp.float32) + b_ref[...]


def _tc_out_proj(x, W_out, b_out):
    N, Lq, C = x.shape
    grid = (N, Lq // BT)
    return pl.pallas_call(
        _proj_body,
        grid=grid,
        in_specs=[
            pl.BlockSpec((1, BT, C), lambda n, i: (n, i, 0)),
            pl.BlockSpec((C, C), lambda n, i: (0, 0)),
            pl.BlockSpec((1, C), lambda n, i: (0, 0)),
        ],
        out_specs=pl.BlockSpec((1, BT, C), lambda n, i: (n, i, 0)),
        out_shape=jax.ShapeDtypeStruct((N, Lq, C), jnp.float32),
    )(x, W_out, b_out.reshape(1, -1))


def _sc_sample(val_t, iw_t):
    N = val_t.shape[0]
    nblk = LQ_PAD // BQ
    mesh = plsc.VectorSubcoreMesh(core_axis_name="c", subcore_axis_name="s")

    @functools.partial(
        pl.kernel,
        out_type=jax.ShapeDtypeStruct((NCHUNK, 16, LQ_PAD), jnp.float32),
        mesh=mesh,
        compiler_params=pltpu.CompilerParams(needs_layout_passes=False),
        scratch_types=[
            pltpu.VMEM((8 * LQ_PAD,), jnp.int32),
            pltpu.VMEM((64, BQ), jnp.int32),
            pltpu.VMEM((16, BQ), jnp.float32),
        ],
    )
    def sc_kernel(val_hbm, iw_hbm, out_hbm, tbl, iwb, outb):
        wid = lax.axis_index("s") * 2 + lax.axis_index("c")
        n = wid // 16
        cg = wid % 16
        m = cg // 2

        for p in range(8):
            pltpu.sync_copy(val_hbm.at[n, cg * 8 + p],
                            tbl.at[pl.ds(p * LQ_PAD, LQ_PAD)])

        def qb_body(qb, _):
            base = qb * BQ
            for c4 in range(4):
                start = pl.multiple_of(c4 * 128 + m * 16, 8)
                pltpu.sync_copy(
                    iw_hbm.at[n, pl.ds(start, 16), pl.ds(base, BQ)],
                    iwb.at[pl.ds(c4 * 16, 16), :])

            def g_body(g, _):
                def j_body(j, accs):
                    iwv = iwb[j, pl.ds(g * 16, 16)]
                    idxv = iwv >> 16
                    wv = plsc.bitcast(iwv << 16, jnp.float32)
                    new = []
                    for p in range(8):
                        vi = plsc.load_gather(tbl, [idxv + p * LQ_PAD])
                        lo = plsc.bitcast(vi << 16, jnp.float32)
                        hi = plsc.bitcast(vi & jnp.int32(-65536), jnp.float32)
                        new.append(accs[2 * p] + wv * lo)
                        new.append(accs[2 * p + 1] + wv * hi)
                    return tuple(new)

                accs = lax.fori_loop(
                    0, 64, j_body,
                    tuple(jnp.zeros((16,), jnp.float32) for _ in range(16)))
                for cc in range(16):
                    outb[cc, pl.ds(g * 16, 16)] = accs[cc]
                return 0

            lax.fori_loop(0, BQ // 16, g_body, 0)
            pltpu.sync_copy(outb, out_hbm.at[wid, :, pl.ds(base, BQ)])
            return 0

        lax.fori_loop(0, nblk, qb_body, 0)

    return sc_kernel(val_t, iw_t)


def kernel(query, reference_points, input_flatten, input_spatial_shapes,
           input_level_start_index, W_val, b_val, W_off, b_off,
           W_attn, b_attn, W_out, b_out):
    N, Lq, C = query.shape
    pad = ((0, 0), (0, 0), (0, LQ_PAD - Lq))
    qt = jnp.pad(query.transpose(0, 2, 1), pad)
    xt = jnp.pad(input_flatten.transpose(0, 2, 1), pad)
    rpt = jnp.pad(reference_points.reshape(N, Lq, 8).transpose(0, 2, 1), pad)
    iw_t, val_t = _tc_prelude(qt, xt, rpt, W_off, b_off, W_attn, b_attn,
                              W_val, b_val)
    out_t = _sc_sample(val_t, iw_t)
    sampled = out_t.reshape(N, 16, 16, LQ_PAD).transpose(0, 3, 1, 2)
    sampled = sampled.reshape(N, LQ_PAD, C)[:, :Lq]
    return _tc_out_proj(sampled, W_out, b_out)


# speedup vs baseline: 1.2818x; 1.1335x over previous
import functools

import jax
import jax.numpy as jnp
import numpy as np
from jax import lax
from jax.experimental import pallas as pl
from jax.experimental.pallas import tpu as pltpu
from jax.experimental.pallas import tpu_sc as plsc

D_MODEL = 256
N_LEVELS = 4
N_HEADS = 8
N_POINTS = 4
SHAPES_PY = [(64, 64), (32, 32), (16, 16), (8, 8)]
LEN_IN = sum(h * w for h, w in SHAPES_PY)
LSI = [0]
for _h, _w in SHAPES_PY[:-1]:
    LSI.append(LSI[-1] + _h * _w)

BT = 680
BT2 = 512
BQ = 512
LQ_PAD = 5632
NCHUNK = 32


def _consts():
    M, P = N_HEADS, N_POINTS
    sel_rp = np.zeros((256, 8), np.float32)
    wh256 = np.zeros((256, 1), np.float32)
    for u in range(256):
        comp, rem = divmod(u, 128)
        m, lp = divmod(rem, 16)
        l = lp // P
        sel_rp[u, l * 2 + comp] = 1.0
        wh256[u, 0] = float(SHAPES_PY[l][1] if comp == 0 else SHAPES_PY[l][0])
    wv = np.zeros((128, 1), np.float32)
    hv = np.zeros((128, 1), np.float32)
    lsiv = np.zeros((128, 1), np.float32)
    for r in range(128):
        l = (r % 16) // P
        wv[r, 0] = float(SHAPES_PY[l][1])
        hv[r, 0] = float(SHAPES_PY[l][0])
        lsiv[r, 0] = float(LSI[l])
    bd = np.kron(np.eye(N_HEADS, dtype=np.float32), np.ones((16, 16), np.float32))
    return tuple(jnp.asarray(a) for a in (sel_rp, wh256, wv, hv, lsiv, bd))


def _off_perm():
    perm = []
    for comp in range(2):
        for m in range(N_HEADS):
            for lp in range(16):
                perm.append(m * 32 + lp * 2 + comp)
    return np.asarray(perm, np.int64)


def _bf16_bits(x):
    b = jax.lax.bitcast_convert_type(x, jnp.int32)
    return (b + 0x7FFF + ((b >> 16) & 1)) >> 16


def _bf16_sbits(x):
    b = jax.lax.bitcast_convert_type(x, jnp.int32)
    return ((b + 0x7FFF + ((b >> 16) & 1)) >> 16) & 0xFFFF


def _prelude_body(qt_ref, xt_ref, rpt_ref, wo_ref, bo_ref, wa_ref, ba_ref,
                  wve_ref, bve_ref, wvo_ref, bvo_ref,
                  selrp_ref, wh_ref, wv_ref, hv_ref, lsi_ref, bd_ref,
                  iw_ref, val_ref):
    qt = qt_ref[0]
    xt = xt_ref[0]
    rpt = rpt_ref[0]
    dot = lambda a, b: jnp.dot(a, b, preferred_element_type=jnp.float32)

    vlo = dot(wve_ref[...], xt) + bve_ref[...]
    vhi = dot(wvo_ref[...], xt) + bvo_ref[...]
    val_ref[0] = _bf16_sbits(vlo) | (_bf16_sbits(vhi) << 16)

    offt = dot(wo_ref[...], qt) + bo_ref[...]
    z = dot(wa_ref[...], qt) + ba_ref[...]
    e = jnp.exp(z)
    av = e / dot(bd_ref[...], e)

    rpx = jnp.dot(selrp_ref[...], rpt, preferred_element_type=jnp.float32,
                  precision=jax.lax.Precision.HIGHEST)
    g = rpx * wh_ref[...] + offt - 0.5
    gx = g[:128]
    gy = g[128:]

    wl = wv_ref[...]
    hl = hv_ref[...]
    x0 = jnp.floor(gx)
    y0 = jnp.floor(gy)
    fx = gx - x0
    fy = gy - y0
    wx1 = fx
    wx0 = 1.0 - fx
    wy1 = fy
    wy0 = 1.0 - fy
    for c4, (dy, dx) in enumerate(((0., 0.), (0., 1.), (1., 0.), (1., 1.))):
        xf = x0 + dx
        yf = y0 + dy
        wx = wx1 if dx else wx0
        wy = wy1 if dy else wy0
        valid = ((xf >= 0.0) & (xf <= wl - 1.0) & (yf >= 0.0)
                 & (yf <= hl - 1.0)).astype(jnp.float32)
        xi = jnp.clip(xf, 0.0, wl - 1.0)
        yi = jnp.clip(yf, 0.0, hl - 1.0)
        rows = (lsi_ref[...] + yi * wl + xi).astype(jnp.int32)
        w = av * wx * wy * valid
        iw_ref[0, c4 * 128:(c4 + 1) * 128] = (rows << 16) | _bf16_bits(w)


def _tc_prelude(qt, xt, rpt, W_off, b_off, W_attn, b_attn, W_val, b_val):
    N = qt.shape[0]
    consts = _consts()
    grid = (N, LQ_PAD // BT2)
    full = lambda shape: pl.BlockSpec(shape, lambda n, i: (0,) * len(shape))
    blk = lambda r: pl.BlockSpec((1, r, BT2), lambda n, i: (n, 0, i))
    col = lambda v: v.reshape(-1, 1)
    return pl.pallas_call(
        _prelude_body,
        grid=grid,
        in_specs=[
            blk(256), blk(256), blk(8),
            full((256, 256)), full((256, 1)),
            full((128, 256)), full((128, 1)),
            full((128, 256)), full((128, 1)),
            full((128, 256)), full((128, 1)),
            full((256, 8)), full((256, 1)),
            full((128, 1)), full((128, 1)), full((128, 1)),
            full((128, 128)),
        ],
        out_specs=[blk(512), blk(128)],
        out_shape=[
            jax.ShapeDtypeStruct((N, 512, LQ_PAD), jnp.int32),
            jax.ShapeDtypeStruct((N, 128, LQ_PAD), jnp.int32),
        ],
    )(qt, xt, rpt,
      W_off[:, _off_perm()].T, col(b_off[_off_perm()]),
      W_attn.T, col(b_attn),
      W_val[:, 0::2].T, col(b_val[0::2]), W_val[:, 1::2].T, col(b_val[1::2]),
      *consts)


def _proj_body(x_ref, w_ref, b_ref, o_ref):
    o_ref[0] = jnp.dot(x_ref[0], w_ref[...],
                       preferred_element_type=jnp.float32) + b_ref[...]


def _tc_out_proj(x, W_out, b_out):
    N, Lq, C = x.shape
    grid = (N, Lq // BT)
    return pl.pallas_call(
        _proj_body,
        grid=grid,
        in_specs=[
            pl.BlockSpec((1, BT, C), lambda n, i: (n, i, 0)),
            pl.BlockSpec((C, C), lambda n, i: (0, 0)),
            pl.BlockSpec((1, C), lambda n, i: (0, 0)),
        ],
        out_specs=pl.BlockSpec((1, BT, C), lambda n, i: (n, i, 0)),
        out_shape=jax.ShapeDtypeStruct((N, Lq, C), jnp.float32),
    )(x, W_out, b_out.reshape(1, -1))


def _sc_sample(val_t, iw_t):
    N = val_t.shape[0]
    nblk = LQ_PAD // BQ
    mesh = plsc.VectorSubcoreMesh(core_axis_name="c", subcore_axis_name="s")

    @functools.partial(
        pl.kernel,
        out_type=jax.ShapeDtypeStruct((NCHUNK, 16, LQ_PAD), jnp.float32),
        mesh=mesh,
        compiler_params=pltpu.CompilerParams(needs_layout_passes=False),
        scratch_types=[
            pltpu.VMEM((8 * LQ_PAD,), jnp.int32),
            pltpu.VMEM((64, BQ), jnp.int32),
            pltpu.VMEM((16, BQ), jnp.float32),
        ],
    )
    def sc_kernel(val_hbm, iw_hbm, out_hbm, tbl, iwb, outb):
        wid = lax.axis_index("s") * 2 + lax.axis_index("c")
        n = wid // 16
        cg = wid % 16
        m = cg // 2

        for p in range(8):
            pltpu.sync_copy(val_hbm.at[n, cg * 8 + p],
                            tbl.at[pl.ds(p * LQ_PAD, LQ_PAD)])

        def qb_body(qb, _):
            base = qb * BQ
            for c4 in range(4):
                start = pl.multiple_of(c4 * 128 + m * 16, 8)
                pltpu.sync_copy(
                    iw_hbm.at[n, pl.ds(start, 16), pl.ds(base, BQ)],
                    iwb.at[pl.ds(c4 * 16, 16), :])

            def g_body(g, _):
                def one(j, accs):
                    iwv = iwb[j, pl.ds(g * 16, 16)]
                    idxv = iwv >> 16
                    wv = plsc.bitcast(iwv << 16, jnp.float32)
                    new = []
                    for p in range(8):
                        vi = plsc.load_gather(
                            tbl.at[pl.ds(p * LQ_PAD, LQ_PAD)], [idxv])
                        lo = plsc.bitcast(vi << 16, jnp.float32)
                        hi = plsc.bitcast(vi, jnp.float32)
                        new.append(accs[2 * p] + wv * lo)
                        new.append(accs[2 * p + 1] + wv * hi)
                    return tuple(new)

                def j_body(j2, accs):
                    return one(j2 + j2 + 1, one(j2 + j2, accs))

                accs = lax.fori_loop(
                    0, 32, j_body,
                    tuple(jnp.zeros((16,), jnp.float32) for _ in range(16)))
                for cc in range(16):
                    outb[cc, pl.ds(g * 16, 16)] = accs[cc]
                return 0

            lax.fori_loop(0, BQ // 16, g_body, 0)
            pltpu.sync_copy(outb, out_hbm.at[wid, :, pl.ds(base, BQ)])
            return 0

        lax.fori_loop(0, nblk, qb_body, 0)

    return sc_kernel(val_t, iw_t)


def kernel(query, reference_points, input_flatten, input_spatial_shapes,
           input_level_start_index, W_val, b_val, W_off, b_off,
           W_attn, b_attn, W_out, b_out):
    N, Lq, C = query.shape
    pad = ((0, 0), (0, 0), (0, LQ_PAD - Lq))
    qt = jnp.pad(query.transpose(0, 2, 1), pad)
    xt = jnp.pad(input_flatten.transpose(0, 2, 1), pad)
    rpt = jnp.pad(reference_points.reshape(N, Lq, 8).transpose(0, 2, 1), pad)
    iw_t, val_t = _tc_prelude(qt, xt, rpt, W_off, b_off, W_attn, b_attn,
                              W_val, b_val)
    out_t = _sc_sample(val_t, iw_t)
    sampled = out_t.reshape(N, 16, 16, LQ_PAD).transpose(0, 3, 1, 2)
    sampled = sampled.reshape(N, LQ_PAD, C)[:, :Lq]
    return _tc_out_proj(sampled, W_out, b_out)
